# baseline (speedup 1.0000x reference)
import functools

import jax
import jax.numpy as jnp
from jax import lax
from jax.experimental import pallas as pl
from jax.experimental.pallas import tpu as pltpu
from jax.experimental.pallas import tpu_sc as plsc

E = 8
TOP_K = 2
D = 1024
FF = 2048
ALPHA = 1.702
LIMIT = 7.0

NW = 32


def _sc_gather(x_rows, src_tok):
    R = src_tok.shape[0]
    W = x_rows.shape[1]
    rows_per_w = R // NW
    CH = 32
    n_ch = rows_per_w // CH
    mesh = plsc.VectorSubcoreMesh(core_axis_name="c", subcore_axis_name="s")

    @functools.partial(
        pl.kernel,
        mesh=mesh,
        out_type=jax.ShapeDtypeStruct((R, W), jnp.int32),
        scratch_types=[
            pltpu.VMEM((rows_per_w,), jnp.int32),
            pltpu.VMEM((CH, W), jnp.int32),
            pltpu.VMEM((CH, W), jnp.int32),
            pltpu.SemaphoreType.DMA,
            pltpu.SemaphoreType.DMA,
        ],
    )
    def k(x_hbm, idx_hbm, out_hbm, idx_v, buf0, buf1, sem0, sem1):
        wid = lax.axis_index("s") * 2 + lax.axis_index("c")
        base = wid * rows_per_w
        pltpu.sync_copy(idx_hbm.at[pl.ds(base, rows_per_w)], idx_v)
        bufs = (buf0, buf1)
        sems = (sem0, sem1)
        pending = pltpu.async_copy(x_hbm.at[idx_v.at[pl.ds(0, CH)]], buf0, sem0)
        for i in range(n_ch):
            if i + 1 < n_ch:
                nxt = pltpu.async_copy(
                    x_hbm.at[idx_v.at[pl.ds((i + 1) * CH, CH)]],
                    bufs[(i + 1) % 2], sems[(i + 1) % 2])
            pending.wait()
            pltpu.sync_copy(bufs[i % 2], out_hbm.at[pl.ds(base + i * CH, CH)])
            if i + 1 < n_ch:
                pending = nxt

    return k(x_rows, src_tok)


def _sc_combine(outs, r0, r1, w0, w1):
    T = r0.shape[0]
    tok_per_w = T // NW
    CH = 32
    n_ch = tok_per_w // CH
    nvec = D // 16
    mesh = plsc.VectorSubcoreMesh(core_axis_name="c", subcore_axis_name="s")

    @functools.partial(
        pl.kernel,
        mesh=mesh,
        out_type=jax.ShapeDtypeStruct((T, D), jnp.float32),
        scratch_types=[
            pltpu.VMEM((tok_per_w,), jnp.int32),
            pltpu.VMEM((tok_per_w,), jnp.int32),
            pltpu.VMEM((tok_per_w, 16), jnp.float32),
            pltpu.VMEM((tok_per_w, 16), jnp.float32),
            pltpu.VMEM((CH, D), jnp.float32),
            pltpu.VMEM((CH, D), jnp.float32),
            pltpu.SemaphoreType.DMA,
            pltpu.SemaphoreType.DMA,
        ],
    )
    def k(outs_hbm, r0_hbm, r1_hbm, w0_hbm, w1_hbm, y_hbm,
          i0_v, i1_v, w0_v, w1_v, bufa, bufb, sem0, sem1):
        wid = lax.axis_index("s") * 2 + lax.axis_index("c")
        base = wid * tok_per_w
        pltpu.sync_copy(r0_hbm.at[pl.ds(base, tok_per_w)], i0_v)
        pltpu.sync_copy(r1_hbm.at[pl.ds(base, tok_per_w)], i1_v)
        pltpu.sync_copy(w0_hbm.at[pl.ds(base, tok_per_w)], w0_v)
        pltpu.sync_copy(w1_hbm.at[pl.ds(base, tok_per_w)], w1_v)

        def chunk_body(cidx, carry):
            off = base + cidx * CH
            cpa = pltpu.async_copy(
                outs_hbm.at[i0_v.at[pl.ds(cidx * CH, CH)]], bufa, sem0)
            cpb = pltpu.async_copy(
                outs_hbm.at[i1_v.at[pl.ds(cidx * CH, CH)]], bufb, sem1)
            cpa.wait()
            cpb.wait()

            def tok_body(i, c2):
                wa = w0_v[cidx * CH + i, :]
                wb = w1_v[cidx * CH + i, :]

                def col_body(j, c3):
                    for u in range(4):
                        sl = pl.ds((j * 4 + u) * 16, 16)
                        a = bufa[i, sl]
                        b = bufb[i, sl]
                        bufa[i, sl] = a * wa + b * wb
                    return c3

                lax.fori_loop(0, nvec // 4, col_body, 0)
                return c2

            lax.fori_loop(0, CH, tok_body, 0)
            pltpu.sync_copy(bufa, y_hbm.at[pl.ds(off, CH)])
            return carry

        lax.fori_loop(0, n_ch, chunk_body, 0)

    return k(outs, r0, r1, w0, w1)


def _tc_mlp(gathered, w1b, b1, w2d, b2, cap):
    CAP_BLK = 512
    F2 = 2 * FF
    F2_BLK = 1024
    CB = cap // CAP_BLK
    NFB = F2 // F2_BLK
    R = E * cap

    def body(xg_ref, w1_ref, b1_ref, w2_ref, b2_ref, out_ref):
        fb = pl.program_id(2)
        x = xg_ref[...]
        b1_blk = b1_ref[0, :, pl.ds(fb * F2_BLK, F2_BLK)]
        gu = jnp.dot(x, w1_ref[0], preferred_element_type=jnp.float32) + b1_blk
        g = jnp.minimum(gu, LIMIT)
        glu = g * jax.nn.sigmoid(g * ALPHA)
        up1 = jnp.clip(gu, -LIMIT, LIMIT) + 1.0
        glu_sh = jnp.roll(glu, 1, axis=1)
        parity = jax.lax.broadcasted_iota(jnp.int32, gu.shape, 1) % 2
        q = jnp.where(parity == 1, up1 * glu_sh, 0.0).astype(jnp.bfloat16)
        part = jnp.dot(q, w2_ref[0], preferred_element_type=jnp.float32)

        @pl.when(fb == 0)
        def _():
            out_ref[...] = part + b2_ref[0]

        @pl.when(fb != 0)
        def _():
            out_ref[...] += part

    grid = (E, CB, NFB)
    return pl.pallas_call(
        body,
        grid=grid,
        in_specs=[
            pl.BlockSpec((CAP_BLK, D), lambda e, cb, fb: (e * CB + cb, 0)),
            pl.BlockSpec((1, D, F2_BLK), lambda e, cb, fb: (e, 0, fb)),
            pl.BlockSpec((1, 1, F2), lambda e, cb, fb: (e, 0, 0)),
            pl.BlockSpec((1, F2_BLK, D), lambda e, cb, fb: (e, fb, 0)),
            pl.BlockSpec((1, 1, D), lambda e, cb, fb: (e, 0, 0)),
        ],
        out_specs=pl.BlockSpec((CAP_BLK, D), lambda e, cb, fb: (e * CB + cb, 0)),
        out_shape=jax.ShapeDtypeStruct((R, D), jnp.float32),
        compiler_params=pltpu.CompilerParams(
            dimension_semantics=("parallel", "parallel", "arbitrary"),
        ),
    )(gathered, w1b, b1, w2d, b2)


def _routing(x_flat, router_weight, router_bias, cap):
    T = x_flat.shape[0]
    logits = x_flat @ router_weight.T + router_bias
    expert_weights, expert_indices = jax.lax.top_k(logits, TOP_K)
    expert_weights = jax.nn.softmax(expert_weights, axis=-1)
    flat_e = expert_indices.reshape(-1)
    wflat = expert_weights.reshape(-1)
    onehot = (flat_e[:, None] == jnp.arange(E, dtype=flat_e.dtype)[None, :]).astype(jnp.int32)
    ends = jnp.cumsum(onehot, axis=0)
    p = jnp.take_along_axis(ends, flat_e[:, None], axis=1)[:, 0] - 1
    kept = p < cap
    rowid = flat_e * cap + p
    rowid_safe = jnp.where(kept, rowid, 0)
    tok_of_assign = jnp.arange(T * TOP_K, dtype=jnp.int32) // TOP_K
    scatter_idx = jnp.where(kept, rowid, E * cap)
    src_tok = jnp.zeros((E * cap,), jnp.int32).at[scatter_idx].set(tok_of_assign)
    w_eff = jnp.where(kept, wflat, 0.0)
    r0 = rowid_safe[0::TOP_K]
    r1 = rowid_safe[1::TOP_K]
    w0 = w_eff[0::TOP_K]
    w1 = w_eff[1::TOP_K]
    return src_tok, r0, r1, w0, w1


def kernel(x, router_weight, router_bias, w1, w1_bias, w2, w2_bias):
    in_shape = x.shape
    T = x.shape[0] * x.shape[1]
    cap = TOP_K * T // E
    x_flat = x.reshape(T, D)

    src_tok, r0, r1, w0, w1c = _routing(x_flat, router_weight, router_bias, cap)

    w1b = w1.astype(jnp.bfloat16)
    w2d = jnp.repeat(w2, 2, axis=1).astype(jnp.bfloat16)
    b1 = w1_bias.reshape(E, 1, 2 * FF)
    b2 = w2_bias.reshape(E, 1, D)

    wc0 = jnp.broadcast_to(w0[:, None], (T, 16))
    wc1 = jnp.broadcast_to(w1c[:, None], (T, 16))

    x16 = x_flat.astype(jnp.bfloat16)
    x_rows = jax.lax.bitcast_convert_type(x16.reshape(T, D // 2, 2), jnp.int32)
    gathered_i32 = _sc_gather(x_rows, src_tok)
    gathered = jax.lax.bitcast_convert_type(
        gathered_i32, jnp.bfloat16).reshape(E * cap, D)
    outs = _tc_mlp(gathered, w1b, b1, w2d, b2, cap)
    y = _sc_combine(outs, r0, r1, wc0, wc1)
    return y.reshape(in_shape)


# speedup vs baseline: 3.4020x; 3.4020x over previous
import functools

import jax
import jax.numpy as jnp
from jax import lax
from jax.experimental import pallas as pl
from jax.experimental.pallas import tpu as pltpu
from jax.experimental.pallas import tpu_sc as plsc

E = 8
TOP_K = 2
D = 1024
FF = 2048
ALPHA = 1.702
LIMIT = 7.0

NW = 32


def _sc_gather(x_rows, src_tok):
    R = src_tok.shape[0]
    W = x_rows.shape[1]
    rows_per_w = R // NW
    CH = 32
    n_ch = rows_per_w // CH
    mesh = plsc.VectorSubcoreMesh(core_axis_name="c", subcore_axis_name="s")

    @functools.partial(
        pl.kernel,
        mesh=mesh,
        out_type=jax.ShapeDtypeStruct((R, W), jnp.float32),
        scratch_types=[
            pltpu.VMEM((rows_per_w,), jnp.int32),
            pltpu.VMEM((CH, W), jnp.float32),
            pltpu.VMEM((CH, W), jnp.float32),
            pltpu.SemaphoreType.DMA,
            pltpu.SemaphoreType.DMA,
        ],
    )
    def k(x_hbm, idx_hbm, out_hbm, idx_v, buf0, buf1, sem0, sem1):
        wid = lax.axis_index("s") * 2 + lax.axis_index("c")
        base = wid * rows_per_w
        pltpu.sync_copy(idx_hbm.at[pl.ds(base, rows_per_w)], idx_v)
        bufs = (buf0, buf1)
        sems = (sem0, sem1)
        pending = pltpu.async_copy(x_hbm.at[idx_v.at[pl.ds(0, CH)]], buf0, sem0)
        for i in range(n_ch):
            if i + 1 < n_ch:
                nxt = pltpu.async_copy(
                    x_hbm.at[idx_v.at[pl.ds((i + 1) * CH, CH)]],
                    bufs[(i + 1) % 2], sems[(i + 1) % 2])
            pending.wait()
            pltpu.sync_copy(bufs[i % 2], out_hbm.at[pl.ds(base + i * CH, CH)])
            if i + 1 < n_ch:
                pending = nxt

    return k(x_rows, src_tok)


def _sc_combine(outs, r0, r1, w0, w1):
    T = r0.shape[0]
    tok_per_w = T // NW
    CH = 32
    n_ch = tok_per_w // CH
    nvec = D // 16
    mesh = plsc.VectorSubcoreMesh(core_axis_name="c", subcore_axis_name="s")

    @functools.partial(
        pl.kernel,
        mesh=mesh,
        out_type=jax.ShapeDtypeStruct((T, D), jnp.float32),
        scratch_types=[
            pltpu.VMEM((tok_per_w,), jnp.int32),
            pltpu.VMEM((tok_per_w,), jnp.int32),
            pltpu.VMEM((tok_per_w, 16), jnp.float32),
            pltpu.VMEM((tok_per_w, 16), jnp.float32),
            pltpu.VMEM((CH, D), jnp.float32),
            pltpu.VMEM((CH, D), jnp.float32),
            pltpu.SemaphoreType.DMA,
            pltpu.SemaphoreType.DMA,
        ],
    )
    def k(outs_hbm, r0_hbm, r1_hbm, w0_hbm, w1_hbm, y_hbm,
          i0_v, i1_v, w0_v, w1_v, bufa, bufb, sem0, sem1):
        wid = lax.axis_index("s") * 2 + lax.axis_index("c")
        base = wid * tok_per_w
        pltpu.sync_copy(r0_hbm.at[pl.ds(base, tok_per_w)], i0_v)
        pltpu.sync_copy(r1_hbm.at[pl.ds(base, tok_per_w)], i1_v)
        pltpu.sync_copy(w0_hbm.at[pl.ds(base, tok_per_w)], w0_v)
        pltpu.sync_copy(w1_hbm.at[pl.ds(base, tok_per_w)], w1_v)

        def chunk_body(cidx, carry):
            off = base + cidx * CH
            cpa = pltpu.async_copy(
                outs_hbm.at[i0_v.at[pl.ds(cidx * CH, CH)]], bufa, sem0)
            cpb = pltpu.async_copy(
                outs_hbm.at[i1_v.at[pl.ds(cidx * CH, CH)]], bufb, sem1)
            cpa.wait()
            cpb.wait()

            def tok_body(i, c2):
                wa = w0_v[cidx * CH + i, :]
                wb = w1_v[cidx * CH + i, :]

                def col_body(j, c3):
                    for u in range(4):
                        sl = pl.ds((j * 4 + u) * 16, 16)
                        a = bufa[i, sl]
                        b = bufb[i, sl]
                        bufa[i, sl] = a * wa + b * wb
                    return c3

                lax.fori_loop(0, nvec // 4, col_body, 0)
                return c2

            lax.fori_loop(0, CH, tok_body, 0)
            pltpu.sync_copy(bufa, y_hbm.at[pl.ds(off, CH)])
            return carry

        lax.fori_loop(0, n_ch, chunk_body, 0)

    return k(outs, r0, r1, w0, w1)


def _tc_mlp(gathered, w1b, b1, w2d, b2, cap):
    CAP_BLK = 512
    F2 = 2 * FF
    F2_BLK = 1024
    CB = cap // CAP_BLK
    NFB = F2 // F2_BLK
    R = E * cap

    def body(xg_ref, w1_ref, b1_ref, w2_ref, b2_ref, out_ref):
        fb = pl.program_id(2)
        x = xg_ref[...].astype(jnp.bfloat16)
        b1_blk = b1_ref[0, :, pl.ds(fb * F2_BLK, F2_BLK)]
        gu = jnp.dot(x, w1_ref[0], preferred_element_type=jnp.float32) + b1_blk
        g = jnp.minimum(gu, LIMIT)
        glu = g * jax.nn.sigmoid(g * ALPHA)
        up1 = jnp.clip(gu, -LIMIT, LIMIT) + 1.0
        glu_sh = jnp.roll(glu, 1, axis=1)
        parity = jax.lax.broadcasted_iota(jnp.int32, gu.shape, 1) % 2
        q = jnp.where(parity == 1, up1 * glu_sh, 0.0).astype(jnp.bfloat16)
        part = jnp.dot(q, w2_ref[0], preferred_element_type=jnp.float32)

        @pl.when(fb == 0)
        def _():
            out_ref[...] = part + b2_ref[0]

        @pl.when(fb != 0)
        def _():
            out_ref[...] += part

    grid = (E, CB, NFB)
    return pl.pallas_call(
        body,
        grid=grid,
        in_specs=[
            pl.BlockSpec((CAP_BLK, D), lambda e, cb, fb: (e * CB + cb, 0)),
            pl.BlockSpec((1, D, F2_BLK), lambda e, cb, fb: (e, 0, fb)),
            pl.BlockSpec((1, 1, F2), lambda e, cb, fb: (e, 0, 0)),
            pl.BlockSpec((1, F2_BLK, D), lambda e, cb, fb: (e, fb, 0)),
            pl.BlockSpec((1, 1, D), lambda e, cb, fb: (e, 0, 0)),
        ],
        out_specs=pl.BlockSpec((CAP_BLK, D), lambda e, cb, fb: (e * CB + cb, 0)),
        out_shape=jax.ShapeDtypeStruct((R, D), jnp.float32),
        compiler_params=pltpu.CompilerParams(
            dimension_semantics=("parallel", "parallel", "arbitrary"),
        ),
    )(gathered, w1b, b1, w2d, b2)


def _routing(x_flat, router_weight, router_bias, cap):
    T = x_flat.shape[0]
    logits = x_flat @ router_weight.T + router_bias
    expert_weights, expert_indices = jax.lax.top_k(logits, TOP_K)
    expert_weights = jax.nn.softmax(expert_weights, axis=-1)
    flat_e = expert_indices.reshape(-1)
    wflat = expert_weights.reshape(-1)
    onehot = (flat_e[:, None] == jnp.arange(E, dtype=flat_e.dtype)[None, :]).astype(jnp.int32)
    ends = jnp.cumsum(onehot, axis=0)
    p = jnp.take_along_axis(ends, flat_e[:, None], axis=1)[:, 0] - 1
    kept = p < cap
    rowid = flat_e * cap + p
    rowid_safe = jnp.where(kept, rowid, 0)
    tok_of_assign = jnp.arange(T * TOP_K, dtype=jnp.int32) // TOP_K
    scatter_idx = jnp.where(kept, rowid, E * cap)
    src_tok = jnp.zeros((E * cap,), jnp.int32).at[scatter_idx].set(tok_of_assign)
    w_eff = jnp.where(kept, wflat, 0.0)
    r0 = rowid_safe[0::TOP_K]
    r1 = rowid_safe[1::TOP_K]
    w0 = w_eff[0::TOP_K]
    w1 = w_eff[1::TOP_K]
    return src_tok, r0, r1, w0, w1


def kernel(x, router_weight, router_bias, w1, w1_bias, w2, w2_bias):
    in_shape = x.shape
    T = x.shape[0] * x.shape[1]
    cap = TOP_K * T // E
    x_flat = x.reshape(T, D)

    src_tok, r0, r1, w0, w1c = _routing(x_flat, router_weight, router_bias, cap)

    w1b = w1.astype(jnp.bfloat16)
    w2d = jnp.repeat(w2, 2, axis=1).astype(jnp.bfloat16)
    b1 = w1_bias.reshape(E, 1, 2 * FF)
    b2 = w2_bias.reshape(E, 1, D)

    wc0 = jnp.broadcast_to(w0[:, None], (T, 16))
    wc1 = jnp.broadcast_to(w1c[:, None], (T, 16))

    gathered = _sc_gather(x_flat, src_tok)
    outs = _tc_mlp(gathered, w1b, b1, w2d, b2, cap)
    y = _sc_combine(outs, r0, r1, wc0, wc1)
    return y.reshape(in_shape)
